# baseline (speedup 1.0000x reference)
import functools

import jax
import jax.numpy as jnp
from jax import lax
from jax.experimental import pallas as pl
from jax.experimental.pallas import tpu as pltpu
from jax.experimental.pallas import tpu_sc as plsc

_LANES = 16
_EPS = 1e-5


def _rsqrt16(v16):
    bits = plsc.bitcast(v16, jnp.int32)
    y = plsc.bitcast(jnp.int32(0x5F3759DF) - (bits >> 1), jnp.float32)
    half = v16 * 0.5
    for _ in range(3):
        y = y * (1.5 - half * y * y)
    return y


def _build_sc_call(N, S, V, D, MAXP):
    info = plsc.get_sparse_core_info()
    NC, NS = info.num_cores, info.num_subcores
    NW = NC * NS
    R = N // NW
    K = 16
    G = R // K
    NVEC = D // _LANES
    assert N % NW == 0 and R % K == 0 and D % _LANES == 0 and S % R == 0

    mesh = plsc.VectorSubcoreMesh(core_axis_name="c", subcore_axis_name="s")

    @functools.partial(
        pl.kernel,
        mesh=mesh,
        out_type=jax.ShapeDtypeStruct((N, D), jnp.float32),
        compiler_params=pltpu.CompilerParams(needs_layout_passes=False),
        scratch_types=[
            pltpu.VMEM((G, K), jnp.int32),
            pltpu.VMEM((2, K, D), jnp.float32),
            pltpu.VMEM((2, K, D), jnp.float32),
            pltpu.VMEM((2, K, D), jnp.float32),
            pltpu.VMEM((D,), jnp.float32),
            pltpu.VMEM((D,), jnp.float32),
            pltpu.SemaphoreType.DMA((2,)),
            pltpu.SemaphoreType.DMA((2,)),
            pltpu.SemaphoreType.DMA((2,)),
        ],
    )
    def emb_kernel(ids_hbm, word_hbm, pos_hbm, gamma_hbm, beta_hbm, out_hbm,
                   ids_v, rows_v, pos_v, outb_v, gamma_v, beta_v,
                   gsem, psem, osem):
        wid = lax.axis_index("s") * NC + lax.axis_index("c")
        base = wid * R
        pos0 = base % S
        crow0 = wid * G
        pltpu.sync_copy(gamma_hbm, gamma_v)
        pltpu.sync_copy(beta_hbm, beta_v)
        pltpu.sync_copy(ids_hbm.at[pl.ds(crow0, G)], ids_v)

        def start_fetch(g, b):
            pltpu.async_copy(word_hbm.at[ids_v.at[g]], rows_v.at[b],
                             gsem.at[b])
            pltpu.async_copy(pos_hbm.at[pl.ds(pos0 + g * K, K)], pos_v.at[b],
                             psem.at[b])

        start_fetch(0, 0)

        def chunk_body(g, carry):
            b = lax.rem(g, 2)
            nb = 1 - b

            @pl.when(g + 1 < G)
            def _():
                start_fetch(g + 1, nb)

            pltpu.make_async_copy(word_hbm.at[ids_v.at[g]], rows_v.at[b],
                                  gsem.at[b]).wait()
            pltpu.make_async_copy(pos_hbm.at[pl.ds(pos0 + g * K, K)],
                                  pos_v.at[b], psem.at[b]).wait()

            @pl.when(g >= 2)
            def _():
                pltpu.make_async_copy(
                    outb_v.at[b], out_hbm.at[pl.ds(base, K)], osem.at[b]
                ).wait()

            m16s, r16s = [], []
            for r in range(K):
                acc_s = jnp.zeros((_LANES,), jnp.float32)
                acc_q = jnp.zeros((_LANES,), jnp.float32)
                for j in range(NVEC):
                    sl = pl.ds(j * _LANES, _LANES)
                    x = rows_v[b, r, sl] + pos_v[b, r, sl]
                    rows_v[b, r, sl] = x
                    acc_s = acc_s + x
                    acc_q = acc_q + x * x
                mean = jnp.sum(acc_s) * (1.0 / D)
                var = jnp.sum(acc_q) * (1.0 / D) - mean * mean
                r16s.append(
                    _rsqrt16(jnp.full((_LANES,), var + _EPS, jnp.float32)))
                m16s.append(jnp.full((_LANES,), mean, jnp.float32))

            def col_body(j, carry2):
                sl = pl.ds(j * _LANES, _LANES)
                g16 = gamma_v[sl]
                b16 = beta_v[sl]
                for r in range(K):
                    x = rows_v[b, r, sl]
                    outb_v[b, r, sl] = (x - m16s[r]) * r16s[r] * g16 + b16
                return carry2

            lax.fori_loop(0, NVEC, col_body, 0)
            pltpu.async_copy(outb_v.at[b],
                             out_hbm.at[pl.ds(base + g * K, K)], osem.at[b])
            return carry

        lax.fori_loop(0, G, chunk_body, 0)

        for tail in (G - 2, G - 1):
            b = tail % 2
            pltpu.make_async_copy(
                outb_v.at[b], out_hbm.at[pl.ds(base, K)], osem.at[b]
            ).wait()

    return emb_kernel


def kernel(input_ids, word_table, pos_table, ln_gamma, ln_beta):
    B, S = input_ids.shape
    V, D = word_table.shape
    MAXP = pos_table.shape[0]
    N = B * S
    K = 16
    ids_2d = input_ids.reshape(N // K, K).astype(jnp.int32)
    call = _build_sc_call(N, S, V, D, MAXP)
    out = call(ids_2d, word_table, pos_table, ln_gamma, ln_beta)
    return out.reshape(B, S, D)


# speedup vs baseline: 1.8615x; 1.8615x over previous
import functools

import jax
import jax.numpy as jnp
from jax import lax
from jax.experimental import pallas as pl
from jax.experimental.pallas import tpu as pltpu
from jax.experimental.pallas import tpu_sc as plsc

_EPS = 1e-5


def _build_gather_call(N, V, D):
    info = plsc.get_sparse_core_info()
    NC, NS = info.num_cores, info.num_subcores
    NW = NC * NS
    R = N // NW
    K = 32
    G = R // K
    assert N % NW == 0 and R % K == 0

    mesh = plsc.VectorSubcoreMesh(core_axis_name="c", subcore_axis_name="s")

    @functools.partial(
        pl.kernel,
        mesh=mesh,
        out_type=jax.ShapeDtypeStruct((N, D), jnp.float32),
        compiler_params=pltpu.CompilerParams(needs_layout_passes=False),
        scratch_types=[
            pltpu.VMEM((G, K), jnp.int32),
            pltpu.VMEM((2, K, D), jnp.float32),
            pltpu.SemaphoreType.DMA((2,)),
            pltpu.SemaphoreType.DMA((2,)),
        ],
    )
    def gather_kernel(ids_hbm, word_hbm, out_hbm, ids_v, rows_v, gsem, osem):
        wid = lax.axis_index("s") * NC + lax.axis_index("c")
        base = wid * R
        pltpu.sync_copy(ids_hbm.at[pl.ds(wid * G, G)], ids_v)

        def start_gather(g, b):
            pltpu.async_copy(word_hbm.at[ids_v.at[g]], rows_v.at[b],
                             gsem.at[b])

        start_gather(0, 0)

        def chunk_body(g, carry):
            b = lax.rem(g, 2)
            nb = 1 - b

            @pl.when(g >= 1)
            def _():
                pltpu.make_async_copy(
                    rows_v.at[nb], out_hbm.at[pl.ds(base, K)], osem.at[nb]
                ).wait()

            @pl.when(g + 1 < G)
            def _():
                start_gather(g + 1, nb)

            pltpu.make_async_copy(word_hbm.at[ids_v.at[g]], rows_v.at[b],
                                  gsem.at[b]).wait()
            pltpu.async_copy(rows_v.at[b],
                             out_hbm.at[pl.ds(base + g * K, K)], osem.at[b])
            return carry

        lax.fori_loop(0, G, chunk_body, 0)
        pltpu.make_async_copy(
            rows_v.at[(G - 1) % 2], out_hbm.at[pl.ds(base, K)],
            osem.at[(G - 1) % 2]
        ).wait()

    return gather_kernel


def _ln_block_kernel(g_ref, p_ref, gamma_ref, beta_ref, o_ref):
    x = g_ref[...] + p_ref[...]
    mean = jnp.mean(x, axis=1, keepdims=True)
    xc = x - mean
    var = jnp.mean(xc * xc, axis=1, keepdims=True)
    o_ref[...] = (xc * lax.rsqrt(var + _EPS)) * gamma_ref[...] + beta_ref[...]


def _build_ln_call(N, S, D, BLK):
    grid = (N // BLK,)
    pos_blocks = S // BLK

    return pl.pallas_call(
        _ln_block_kernel,
        grid=grid,
        in_specs=[
            pl.BlockSpec((BLK, D), lambda i: (i, 0)),
            pl.BlockSpec((BLK, D), lambda i: (lax.rem(i, pos_blocks), 0)),
            pl.BlockSpec((1, D), lambda i: (0, 0)),
            pl.BlockSpec((1, D), lambda i: (0, 0)),
        ],
        out_specs=pl.BlockSpec((BLK, D), lambda i: (i, 0)),
        out_shape=jax.ShapeDtypeStruct((N, D), jnp.float32),
        compiler_params=pltpu.CompilerParams(
            dimension_semantics=("arbitrary",),
        ),
    )


def kernel(input_ids, word_table, pos_table, ln_gamma, ln_beta):
    B, S = input_ids.shape
    V, D = word_table.shape
    N = B * S
    K = 32
    BLK = 256
    ids_2d = input_ids.reshape(N // K, K).astype(jnp.int32)
    gathered = _build_gather_call(N, V, D)(ids_2d, word_table)
    ln = _build_ln_call(N, S, D, BLK)
    out = ln(gathered, pos_table[:S], ln_gamma.reshape(1, D),
             ln_beta.reshape(1, D))
    return out.reshape(B, S, D)


# speedup vs baseline: 1.8827x; 1.0114x over previous
import functools

import jax
import jax.numpy as jnp
from jax import lax
from jax.experimental import pallas as pl
from jax.experimental.pallas import tpu as pltpu
from jax.experimental.pallas import tpu_sc as plsc

_EPS = 1e-5


def _build_gather_call(N, V, D):
    info = plsc.get_sparse_core_info()
    NC, NS = info.num_cores, info.num_subcores
    NW = NC * NS
    R = N // NW
    K = 32
    G = R // K
    assert N % NW == 0 and R % K == 0

    mesh = plsc.VectorSubcoreMesh(core_axis_name="c", subcore_axis_name="s")

    @functools.partial(
        pl.kernel,
        mesh=mesh,
        out_type=jax.ShapeDtypeStruct((N, D), jnp.float32),
        compiler_params=pltpu.CompilerParams(needs_layout_passes=False),
        scratch_types=[
            pltpu.VMEM((G, K), jnp.int32),
            pltpu.VMEM((2, K, D), jnp.float32),
            pltpu.SemaphoreType.DMA((2,)),
            pltpu.SemaphoreType.DMA((2,)),
        ],
    )
    def gather_kernel(ids_hbm, word_hbm, out_hbm, ids_v, rows_v, gsem, osem):
        wid = lax.axis_index("s") * NC + lax.axis_index("c")
        base = wid * R
        pltpu.sync_copy(ids_hbm.at[pl.ds(wid * G, G)], ids_v)

        def start_gather(g, b):
            pltpu.async_copy(word_hbm.at[ids_v.at[g]], rows_v.at[b],
                             gsem.at[b])

        start_gather(0, 0)

        def chunk_body(g, carry):
            b = lax.rem(g, 2)
            nb = 1 - b

            @pl.when(g >= 1)
            def _():
                pltpu.make_async_copy(
                    rows_v.at[nb], out_hbm.at[pl.ds(base, K)], osem.at[nb]
                ).wait()

            @pl.when(g + 1 < G)
            def _():
                start_gather(g + 1, nb)

            pltpu.make_async_copy(word_hbm.at[ids_v.at[g]], rows_v.at[b],
                                  gsem.at[b]).wait()
            pltpu.async_copy(rows_v.at[b],
                             out_hbm.at[pl.ds(base + g * K, K)], osem.at[b])
            return carry

        lax.fori_loop(0, G, chunk_body, 0)
        pltpu.make_async_copy(
            rows_v.at[(G - 1) % 2], out_hbm.at[pl.ds(base, K)],
            osem.at[(G - 1) % 2]
        ).wait()

    return gather_kernel


def _ln_block_kernel(g_ref, p_ref, gamma_ref, beta_ref, o_ref):
    x = g_ref[...] + p_ref[...]
    mean = jnp.mean(x, axis=1, keepdims=True)
    xc = x - mean
    var = jnp.mean(xc * xc, axis=1, keepdims=True)
    o_ref[...] = (xc * lax.rsqrt(var + _EPS)) * gamma_ref[...] + beta_ref[...]


def _build_ln_call(N, S, D, BLK):
    nb = N // S
    pos_blocks = S // BLK
    grid = (pos_blocks, nb)

    return pl.pallas_call(
        _ln_block_kernel,
        grid=grid,
        in_specs=[
            pl.BlockSpec((BLK, D), lambda i, j: (j * pos_blocks + i, 0)),
            pl.BlockSpec((BLK, D), lambda i, j: (i, 0)),
            pl.BlockSpec((1, D), lambda i, j: (0, 0)),
            pl.BlockSpec((1, D), lambda i, j: (0, 0)),
        ],
        out_specs=pl.BlockSpec((BLK, D), lambda i, j: (j * pos_blocks + i, 0)),
        out_shape=jax.ShapeDtypeStruct((N, D), jnp.float32),
        compiler_params=pltpu.CompilerParams(
            dimension_semantics=("arbitrary", "arbitrary"),
        ),
    )


def kernel(input_ids, word_table, pos_table, ln_gamma, ln_beta):
    B, S = input_ids.shape
    V, D = word_table.shape
    N = B * S
    K = 32
    BLK = 256
    ids_2d = input_ids.reshape(N // K, K).astype(jnp.int32)
    gathered = _build_gather_call(N, V, D)(ids_2d, word_table)
    ln = _build_ln_call(N, S, D, BLK)
    out = ln(gathered, pos_table[:S], ln_gamma.reshape(1, D),
             ln_beta.reshape(1, D))
    return out.reshape(B, S, D)


# speedup vs baseline: 1.9461x; 1.0337x over previous
import functools

import jax
import jax.numpy as jnp
from jax import lax
from jax.experimental import pallas as pl
from jax.experimental.pallas import tpu as pltpu
from jax.experimental.pallas import tpu_sc as plsc

_EPS = 1e-5


def _build_gather_call(B, S, V, D):
    N = B * S
    info = plsc.get_sparse_core_info()
    NC, NS = info.num_cores, info.num_subcores
    NW = NC * NS
    R = N // NW
    K = 32
    G = R // K
    WPB = S // R
    assert N % NW == 0 and R % K == 0 and S % R == 0

    mesh = plsc.VectorSubcoreMesh(core_axis_name="c", subcore_axis_name="s")

    @functools.partial(
        pl.kernel,
        mesh=mesh,
        out_type=jax.ShapeDtypeStruct((N, D), jnp.float32),
        compiler_params=pltpu.CompilerParams(needs_layout_passes=False),
        scratch_types=[
            pltpu.VMEM((R,), jnp.int32),
            pltpu.VMEM((2, K, D), jnp.float32),
            pltpu.SemaphoreType.DMA((2,)),
            pltpu.SemaphoreType.DMA((2,)),
        ],
    )
    def gather_kernel(ids_hbm, word_hbm, out_hbm, ids_v, rows_v, gsem, osem):
        wid = lax.axis_index("s") * NC + lax.axis_index("c")
        base = wid * R
        pltpu.sync_copy(
            ids_hbm.at[wid // WPB, pl.ds((wid % WPB) * R, R)], ids_v)

        def start_gather(g, b):
            pltpu.async_copy(word_hbm.at[ids_v.at[pl.ds(g * K, K)]],
                             rows_v.at[b], gsem.at[b])

        start_gather(0, 0)

        def chunk_body(g, carry):
            b = lax.rem(g, 2)
            nb = 1 - b

            @pl.when(g >= 1)
            def _():
                pltpu.make_async_copy(
                    rows_v.at[nb], out_hbm.at[pl.ds(base, K)], osem.at[nb]
                ).wait()

            @pl.when(g + 1 < G)
            def _():
                start_gather(g + 1, nb)

            pltpu.make_async_copy(word_hbm.at[ids_v.at[pl.ds(g * K, K)]],
                                  rows_v.at[b], gsem.at[b]).wait()
            pltpu.async_copy(rows_v.at[b],
                             out_hbm.at[pl.ds(base + g * K, K)], osem.at[b])
            return carry

        lax.fori_loop(0, G, chunk_body, 0)
        pltpu.make_async_copy(
            rows_v.at[(G - 1) % 2], out_hbm.at[pl.ds(base, K)],
            osem.at[(G - 1) % 2]
        ).wait()

    return gather_kernel


def _ln_block_kernel(g_ref, p_ref, gamma_ref, beta_ref, o_ref):
    x = g_ref[...] + p_ref[...]
    mean = jnp.mean(x, axis=1, keepdims=True)
    xc = x - mean
    var = jnp.mean(xc * xc, axis=1, keepdims=True)
    o_ref[...] = (xc * lax.rsqrt(var + _EPS)) * gamma_ref[...] + beta_ref[...]


def _build_ln_call(N, S, D, BLK):
    nb = N // S
    pos_blocks = S // BLK
    grid = (pos_blocks, nb)

    return pl.pallas_call(
        _ln_block_kernel,
        grid=grid,
        in_specs=[
            pl.BlockSpec((BLK, D), lambda i, j: (j * pos_blocks + i, 0)),
            pl.BlockSpec((BLK, D), lambda i, j: (i, 0)),
            pl.BlockSpec((1, D), lambda i, j: (0, 0)),
            pl.BlockSpec((1, D), lambda i, j: (0, 0)),
        ],
        out_specs=pl.BlockSpec((BLK, D), lambda i, j: (j * pos_blocks + i, 0)),
        out_shape=jax.ShapeDtypeStruct((N, D), jnp.float32),
        compiler_params=pltpu.CompilerParams(
            dimension_semantics=("arbitrary", "arbitrary"),
        ),
    )


def kernel(input_ids, word_table, pos_table, ln_gamma, ln_beta):
    B, S = input_ids.shape
    V, D = word_table.shape
    N = B * S
    BLK = 256
    ids = input_ids.astype(jnp.int32)
    gathered = _build_gather_call(B, S, V, D)(ids, word_table)
    ln = _build_ln_call(N, S, D, BLK)
    out = ln(gathered, pos_table, ln_gamma.reshape(1, D),
             ln_beta.reshape(1, D))
    return out.reshape(B, S, D)


# speedup vs baseline: 1.9801x; 1.0175x over previous
import functools

import jax
import jax.numpy as jnp
from jax import lax
from jax.experimental import pallas as pl
from jax.experimental.pallas import tpu as pltpu
from jax.experimental.pallas import tpu_sc as plsc

_EPS = 1e-5


def _build_gather_call(B, S, V, D):
    N = B * S
    info = plsc.get_sparse_core_info()
    NC, NS = info.num_cores, info.num_subcores
    NW = NC * NS
    R = N // NW
    K = 32
    G = R // K
    WPB = S // R
    assert N % NW == 0 and R % K == 0 and S % R == 0

    mesh = plsc.VectorSubcoreMesh(core_axis_name="c", subcore_axis_name="s")

    @functools.partial(
        pl.kernel,
        mesh=mesh,
        out_type=jax.ShapeDtypeStruct((N, D), jnp.float32),
        compiler_params=pltpu.CompilerParams(needs_layout_passes=False),
        scratch_types=[
            pltpu.VMEM((R,), jnp.int32),
            pltpu.VMEM((2, K, D), jnp.float32),
            pltpu.SemaphoreType.DMA((2,)),
            pltpu.SemaphoreType.DMA((2,)),
        ],
    )
    def gather_kernel(ids_hbm, word_hbm, out_hbm, ids_v, rows_v, gsem, osem):
        wid = lax.axis_index("s") * NC + lax.axis_index("c")
        base = wid * R
        pltpu.sync_copy(
            ids_hbm.at[wid // WPB, pl.ds((wid % WPB) * R, R)], ids_v)

        def start_gather(g, b):
            pltpu.async_copy(word_hbm.at[ids_v.at[pl.ds(g * K, K)]],
                             rows_v.at[b], gsem.at[b])

        start_gather(0, 0)

        def chunk_body(g, carry):
            b = lax.rem(g, 2)
            nb = 1 - b

            @pl.when(g >= 1)
            def _():
                pltpu.make_async_copy(
                    rows_v.at[nb], out_hbm.at[pl.ds(base, K)], osem.at[nb]
                ).wait()

            @pl.when(g + 1 < G)
            def _():
                start_gather(g + 1, nb)

            pltpu.make_async_copy(word_hbm.at[ids_v.at[pl.ds(g * K, K)]],
                                  rows_v.at[b], gsem.at[b]).wait()
            pltpu.async_copy(rows_v.at[b],
                             out_hbm.at[pl.ds(base + g * K, K)], osem.at[b])
            return carry

        lax.fori_loop(0, G, chunk_body, 0)
        pltpu.make_async_copy(
            rows_v.at[(G - 1) % 2], out_hbm.at[pl.ds(base, K)],
            osem.at[(G - 1) % 2]
        ).wait()

    return gather_kernel


def _ln_block_kernel(g_ref, p_ref, gamma_ref, beta_ref, o_ref):
    x = g_ref[...] + p_ref[...]
    mean = jnp.mean(x, axis=1, keepdims=True)
    xc = x - mean
    var = jnp.mean(xc * xc, axis=1, keepdims=True)
    o_ref[...] = (xc * lax.rsqrt(var + _EPS)) * gamma_ref[...] + beta_ref[...]


def _ln_chain_kernel(prev_ref, g_ref, p_ref, gamma_ref, beta_ref, o_ref):
    del prev_ref
    _ln_block_kernel(g_ref, p_ref, gamma_ref, beta_ref, o_ref)


def _build_ln_call(N, S, D, BLK, nb_slice, batch_off, chained):
    pos_blocks = S // BLK
    grid = (pos_blocks, nb_slice)

    data_specs = [
        pl.BlockSpec((BLK, D), lambda i, j: (j * pos_blocks + i, 0)),
        pl.BlockSpec((BLK, D), lambda i, j: (i, 0)),
        pl.BlockSpec((1, D), lambda i, j: (0, 0)),
        pl.BlockSpec((1, D), lambda i, j: (0, 0)),
    ]
    if chained:
        in_specs = [pl.BlockSpec(memory_space=pl.ANY)] + data_specs
        body = _ln_chain_kernel
        aliases = {0: 0}
    else:
        in_specs = data_specs
        body = _ln_block_kernel
        aliases = {}

    return pl.pallas_call(
        body,
        grid=grid,
        in_specs=in_specs,
        out_specs=pl.BlockSpec(
            (BLK, D),
            lambda i, j: ((batch_off + j) * pos_blocks + i, 0),
        ),
        out_shape=jax.ShapeDtypeStruct((N, D), jnp.float32),
        input_output_aliases=aliases,
        compiler_params=pltpu.CompilerParams(
            dimension_semantics=("arbitrary", "arbitrary"),
        ),
    )


def kernel(input_ids, word_table, pos_table, ln_gamma, ln_beta):
    B, S = input_ids.shape
    V, D = word_table.shape
    N = B * S
    BLK = 256
    NSLICES = 2
    bs = B // NSLICES
    ids = input_ids.astype(jnp.int32)
    gamma2 = ln_gamma.reshape(1, D)
    beta2 = ln_beta.reshape(1, D)

    gather = _build_gather_call(bs, S, V, D)
    gathered = [gather(ids[s * bs:(s + 1) * bs], word_table)
                for s in range(NSLICES)]

    out = None
    for s in range(NSLICES):
        ln = _build_ln_call(N, S, D, BLK, bs, s * bs, chained=s > 0)
        if s == 0:
            out = ln(gathered[s], pos_table, gamma2, beta2)
        else:
            out = ln(out, gathered[s], pos_table, gamma2, beta2)
    return out.reshape(B, S, D)
